# baseline (speedup 1.0000x reference)
import functools

import jax
import jax.numpy as jnp
import numpy as np
from jax import lax
from jax.experimental import pallas as pl
from jax.experimental.pallas import tpu as pltpu
from jax.experimental.pallas import tpu_sc as plsc

BATCH = 16384
NUM_FINE = 100
GROUP = 5

NC = 2
NS = 16
L = 16
NW = NC * NS
ROWS_PER_W = BATCH // NW
CHUNKS = ROWS_PER_W // L

LN2 = np.float32(0.6931471805599453)
SQRT2 = np.float32(1.4142135623730951)


def _vlog(d):
    bits = plsc.bitcast(d, jnp.int32)
    ex = (bits >> 23) - 127
    mant = plsc.bitcast((bits & 0x007FFFFF) | 0x3F800000, jnp.float32)
    adj = mant > SQRT2
    mant = jnp.where(adj, mant * np.float32(0.5), mant)
    ex = jnp.where(adj, ex + 1, ex)
    z = (mant - np.float32(1.0)) / (mant + np.float32(1.0))
    z2 = z * z
    logm = np.float32(2.0) * z * (
        np.float32(1.0)
        + z2 * (np.float32(1.0 / 3.0)
                + z2 * (np.float32(1.0 / 5.0) + z2 * np.float32(1.0 / 7.0))))
    return logm + ex.astype(jnp.float32) * LN2


def _body(yhatT_hbm, yfine_hbm, out_hbm, yhat_v, yfine_v, acc_v, sem0, sem1):
    wid = lax.axis_index("s") * NC + lax.axis_index("c")
    base = wid * ROWS_PER_W
    half = ROWS_PER_W // 2
    cp0 = pltpu.async_copy(
        yhatT_hbm.at[:, pl.ds(base, half)], yhat_v.at[:, pl.ds(0, half)], sem0)
    cp1 = pltpu.async_copy(
        yhatT_hbm.at[:, pl.ds(base + half, half)],
        yhat_v.at[:, pl.ds(half, half)], sem1)
    pltpu.sync_copy(yfine_hbm.at[pl.ds(base, ROWS_PER_W)], yfine_v)

    lane = lax.iota(jnp.int32, L)

    def chunk(c, acc):
        yf = yfine_v[pl.ds(c * L, L)]
        g = (yf.astype(jnp.float32) * np.float32(0.2)).astype(jnp.int32)
        col0 = (g << 2) + g
        r = yf - col0
        rows = c * L + lane
        xs = [plsc.load_gather(yhat_v, [col0 + j, rows]) for j in range(GROUP)]
        m = xs[0]
        for j in range(1, GROUP):
            m = jnp.maximum(m, xs[j])
        es = [jnp.exp(x - m) for x in xs]
        s = es[0]
        for j in range(1, GROUP):
            s = s + es[j]
        zero = jnp.zeros((L,), jnp.float32)
        er = zero
        t = zero
        for j in range(GROUP):
            is_r = r == j
            er = jnp.where(is_r, es[j], er)
            t = t + jnp.where(is_r, zero, es[j] * (xs[j] - m))
        u = s - er
        d = u + np.float32(1e-7) * s
        return acc + (t - _vlog(d) * u) / d

    acc = jnp.zeros((L,), jnp.float32)
    cp0.wait()
    acc = lax.fori_loop(0, CHUNKS // 2, chunk, acc)
    cp1.wait()
    acc = lax.fori_loop(CHUNKS // 2, CHUNKS, chunk, acc)
    acc_v[...] = acc
    pltpu.sync_copy(acc_v, out_hbm.at[wid])


@jax.jit
def kernel(yHat, y_fine):
    mesh = plsc.VectorSubcoreMesh(core_axis_name="c", subcore_axis_name="s")
    partials = pl.kernel(
        _body,
        mesh=mesh,
        compiler_params=pltpu.CompilerParams(needs_layout_passes=False),
        out_type=jax.ShapeDtypeStruct((NW, L), jnp.float32),
        scratch_types=[
            pltpu.VMEM((NUM_FINE, ROWS_PER_W), jnp.float32),
            pltpu.VMEM((ROWS_PER_W,), jnp.int32),
            pltpu.VMEM((L,), jnp.float32),
            pltpu.SemaphoreType.DMA,
            pltpu.SemaphoreType.DMA,
        ],
    )(yHat.T, y_fine.astype(jnp.int32))
    return jnp.sum(partials) * np.float32(1.0 / (BATCH * GROUP))


# speedup vs baseline: 1.0066x; 1.0066x over previous
import jax
import jax.numpy as jnp
import numpy as np
from jax import lax
from jax.experimental import pallas as pl
from jax.experimental.pallas import tpu as pltpu
from jax.experimental.pallas import tpu_sc as plsc

BATCH = 16384
NUM_FINE = 100
GROUP = 5

NC = 2
NS = 16
L = 16
NW = NC * NS
ROWS_PER_W = BATCH // NW
CHUNKS = ROWS_PER_W // L

LN2 = np.float32(0.6931471805599453)
SQRT2 = np.float32(1.4142135623730951)


def _vlog(d):
    bits = plsc.bitcast(d, jnp.int32)
    ex = (bits >> 23) - 127
    mant = plsc.bitcast((bits & 0x007FFFFF) | 0x3F800000, jnp.float32)
    adj = mant > SQRT2
    mant = jnp.where(adj, mant * np.float32(0.5), mant)
    ex = jnp.where(adj, ex + 1, ex)
    z = (mant - np.float32(1.0)) / (mant + np.float32(1.0))
    z2 = z * z
    logm = np.float32(2.0) * z * (
        np.float32(1.0)
        + z2 * (np.float32(1.0 / 3.0)
                + z2 * (np.float32(1.0 / 5.0) + z2 * np.float32(1.0 / 7.0))))
    return logm + ex.astype(jnp.float32) * LN2


def _body(yhatT_hbm, yfine_hbm, out_hbm, yhat_v, yfine_v, acc_v):
    wid = lax.axis_index("s") * NC + lax.axis_index("c")
    base = wid * ROWS_PER_W
    pltpu.sync_copy(yhatT_hbm.at[:, pl.ds(base, ROWS_PER_W)], yhat_v)
    pltpu.sync_copy(yfine_hbm.at[pl.ds(base, ROWS_PER_W)], yfine_v)

    lane = lax.iota(jnp.int32, L)

    def chunk(c, acc):
        yf = yfine_v[pl.ds(c * L, L)]
        g = (yf.astype(jnp.float32) * np.float32(0.2)).astype(jnp.int32)
        col0 = (g << 2) + g
        r = yf - col0
        rows = c * L + lane
        xs = [plsc.load_gather(yhat_v, [col0 + j, rows]) for j in range(GROUP)]
        m = xs[0]
        for j in range(1, GROUP):
            m = jnp.maximum(m, xs[j])
        es = [jnp.exp(x - m) for x in xs]
        s = es[0]
        for j in range(1, GROUP):
            s = s + es[j]
        zero = jnp.zeros((L,), jnp.float32)
        er = zero
        t = zero
        for j in range(GROUP):
            is_r = r == j
            er = jnp.where(is_r, es[j], er)
            t = t + jnp.where(is_r, zero, es[j] * (xs[j] - m))
        u = s - er
        d = u + np.float32(1e-7) * s
        return acc + (t - _vlog(d) * u) / d

    acc = lax.fori_loop(0, CHUNKS, chunk, jnp.zeros((L,), jnp.float32))
    acc_v[...] = acc
    pltpu.sync_copy(acc_v, out_hbm.at[wid])


@jax.jit
def kernel(yHat, y_fine):
    mesh = plsc.VectorSubcoreMesh(core_axis_name="c", subcore_axis_name="s")
    partials = pl.kernel(
        _body,
        mesh=mesh,
        compiler_params=pltpu.CompilerParams(needs_layout_passes=False),
        out_type=jax.ShapeDtypeStruct((NW, L), jnp.float32),
        scratch_types=[
            pltpu.VMEM((NUM_FINE, ROWS_PER_W), jnp.float32),
            pltpu.VMEM((ROWS_PER_W,), jnp.int32),
            pltpu.VMEM((L,), jnp.float32),
        ],
    )(yHat.T, y_fine.astype(jnp.int32))
    return jnp.sum(partials) * np.float32(1.0 / (BATCH * GROUP))
